# baseline (speedup 1.0000x reference)
import jax
import jax.numpy as jnp
from jax.experimental import pallas as pl
from jax.experimental.pallas import tpu as pltpu

_DIM = 1024
_SCALE = _DIM ** (-0.5)
_CHUNKS = (256, 1536, 2048, 2048, 2048, 256)
_MAX_ROWS = max(_CHUNKS)
_K = 3


def _stream_scale_kernel(emb_hbm, out_hbm, in_slots, out_slots, load_sems,
                         store_sems):
    offs = []
    o = 0
    for r in _CHUNKS:
        offs.append(o)
        o += r
    n = len(_CHUNKS)

    def load(i):
        s = i % _K
        pltpu.make_async_copy(
            emb_hbm.at[pl.ds(offs[i], _CHUNKS[i]), :],
            in_slots.at[s, pl.ds(0, _CHUNKS[i]), :], load_sems.at[s]).start()

    def store_copy(i):
        s = i % _K
        return pltpu.make_async_copy(
            out_slots.at[s, pl.ds(0, _CHUNKS[i]), :],
            out_hbm.at[pl.ds(offs[i], _CHUNKS[i]), :], store_sems.at[s])

    for i in range(min(_K, n)):
        load(i)
    for i in range(n):
        s = i % _K
        pltpu.make_async_copy(
            emb_hbm.at[pl.ds(offs[i], _CHUNKS[i]), :],
            in_slots.at[s, pl.ds(0, _CHUNKS[i]), :], load_sems.at[s]).wait()
        if i >= _K:
            store_copy(i - _K).wait()
        out_slots[s, pl.ds(0, _CHUNKS[i]), :] = (
            in_slots[s, pl.ds(0, _CHUNKS[i]), :] * _SCALE)
        store_copy(i).start()
        if i + _K < n:
            load(i + _K)
    for i in range(max(0, n - _K), n):
        store_copy(i).wait()


def kernel(x, emb):
    seq_len = x.shape[1]
    rows, dim = emb.shape
    assert seq_len == rows and dim == _DIM and sum(_CHUNKS) == rows
    return pl.pallas_call(
        _stream_scale_kernel,
        in_specs=[pl.BlockSpec(memory_space=pl.ANY)],
        out_specs=pl.BlockSpec(memory_space=pl.ANY),
        out_shape=jax.ShapeDtypeStruct((rows, dim), emb.dtype),
        scratch_shapes=[
            pltpu.VMEM((_K, _MAX_ROWS, _DIM), jnp.float32),
            pltpu.VMEM((_K, _MAX_ROWS, _DIM), jnp.float32),
            pltpu.SemaphoreType.DMA((_K,)),
            pltpu.SemaphoreType.DMA((_K,)),
        ],
    )(emb)


# speedup vs baseline: 1.0286x; 1.0286x over previous
import jax
import jax.numpy as jnp
from jax.experimental import pallas as pl
from jax.experimental.pallas import tpu as pltpu

_DIM = 1024
_SCALE = _DIM ** (-0.5)
_CHUNKS = (256, 1792, 2048, 2048, 1792, 256)
_MAX_ROWS = max(_CHUNKS)
_K = 3


def _stream_scale_kernel(emb_hbm, out_hbm, in_slots, out_slots, load_sems,
                         store_sems):
    offs = []
    o = 0
    for r in _CHUNKS:
        offs.append(o)
        o += r
    n = len(_CHUNKS)

    def load(i):
        s = i % _K
        pltpu.make_async_copy(
            emb_hbm.at[pl.ds(offs[i], _CHUNKS[i]), :],
            in_slots.at[s, pl.ds(0, _CHUNKS[i]), :], load_sems.at[s]).start()

    def store_copy(i):
        s = i % _K
        return pltpu.make_async_copy(
            out_slots.at[s, pl.ds(0, _CHUNKS[i]), :],
            out_hbm.at[pl.ds(offs[i], _CHUNKS[i]), :], store_sems.at[s])

    for i in range(min(_K, n)):
        load(i)
    for i in range(n):
        s = i % _K
        pltpu.make_async_copy(
            emb_hbm.at[pl.ds(offs[i], _CHUNKS[i]), :],
            in_slots.at[s, pl.ds(0, _CHUNKS[i]), :], load_sems.at[s]).wait()
        if i >= _K:
            store_copy(i - _K).wait()
        out_slots[s, pl.ds(0, _CHUNKS[i]), :] = (
            in_slots[s, pl.ds(0, _CHUNKS[i]), :] * _SCALE)
        store_copy(i).start()
        if i + _K < n:
            load(i + _K)
    for i in range(max(0, n - _K), n):
        store_copy(i).wait()


def kernel(x, emb):
    seq_len = x.shape[1]
    rows, dim = emb.shape
    assert seq_len == rows and dim == _DIM and sum(_CHUNKS) == rows
    return pl.pallas_call(
        _stream_scale_kernel,
        in_specs=[pl.BlockSpec(memory_space=pl.ANY)],
        out_specs=pl.BlockSpec(memory_space=pl.ANY),
        out_shape=jax.ShapeDtypeStruct((rows, dim), emb.dtype),
        scratch_shapes=[
            pltpu.VMEM((_K, _MAX_ROWS, _DIM), jnp.float32),
            pltpu.VMEM((_K, _MAX_ROWS, _DIM), jnp.float32),
            pltpu.SemaphoreType.DMA((_K,)),
            pltpu.SemaphoreType.DMA((_K,)),
        ],
    )(emb)
